# baseline (speedup 1.0000x reference)
import functools

import jax
import jax.numpy as jnp
from jax import lax
from jax.experimental import pallas as pl
from jax.experimental.pallas import tpu as pltpu
from jax.experimental.pallas import tpu_sc as plsc

_NC = 2
_NS = 16
_NW = _NC * _NS
_LANES = 16


def _sc_gather(x_cat_flat, table_flat, F, V, D):
    R = x_cat_flat.shape[0]
    dt = table_flat.dtype
    r_per_w = R // _NW
    CH = 1664
    n_ch = r_per_w // CH
    n_vec = r_per_w // _LANES

    mesh = plsc.VectorSubcoreMesh(
        core_axis_name="c", subcore_axis_name="s",
        num_cores=_NC, num_subcores=_NS)

    @functools.partial(
        pl.kernel,
        out_type=jax.ShapeDtypeStruct((R, D), dt),
        mesh=mesh,
        scratch_types=[
            pltpu.VMEM((r_per_w,), jnp.int32),
            pltpu.VMEM((CH, D), dt),
            pltpu.VMEM((CH, D), dt),
            pltpu.SemaphoreType.DMA,
            pltpu.SemaphoreType.DMA,
        ],
        compiler_params=pltpu.CompilerParams(use_tc_tiling_on_sc=False),
    )
    def k(xc_hbm, tab_hbm, out_hbm, idx_v, rows_a, rows_b, sem_a, sem_b):
        wid = lax.axis_index("s") * _NC + lax.axis_index("c")
        base = wid * r_per_w
        pltpu.sync_copy(xc_hbm.at[pl.ds(base * 1, r_per_w)], idx_v)

        lane = lax.iota(jnp.int32, _LANES)

        def idx_body(i, carry):
            p = i * _LANES + lane
            f = lax.rem(p, F)
            sl = pl.ds(i * _LANES, _LANES)
            idx_v[sl] = idx_v[sl] + f * V
            return carry

        lax.fori_loop(0, n_vec, idx_body, 0, unroll=4)

        bufs = (rows_a, rows_b)
        sems = (sem_a, sem_b)

        def fire(c, buf, sem):
            return pltpu.async_copy(
                tab_hbm.at[idx_v.at[pl.ds(c * CH, CH)]], buf, sem)

        fire(0, bufs[0], sems[0]).wait()
        for c in range(n_ch):
            if c + 1 < n_ch:
                nxt = fire(c + 1, bufs[(c + 1) % 2], sems[(c + 1) % 2])
            pltpu.sync_copy(bufs[c % 2], out_hbm.at[pl.ds(base + c * CH, CH)])
            if c + 1 < n_ch:
                nxt.wait()

    return k(x_cat_flat, table_flat)


def _mlp_body(BLK, xnum_ref, embs_ref, w1a_ref, w1b_ref, b1_ref, w2_ref,
              b2_ref, wo_ref, bo_ref, gamma_ref, beta_ref, out_ref, stats_ref):
    pid = pl.program_id(0)
    n = xnum_ref.shape[0]

    @pl.when(pid == 0)
    def _():
        x = xnum_ref[...]
        mu = jnp.sum(x, axis=0, keepdims=True) * (1.0 / n)
        var = jnp.sum(x * x, axis=0, keepdims=True) * (1.0 / n) - mu * mu
        s = gamma_ref[...] * lax.rsqrt(var + 1e-5)
        c = beta_ref[...] - mu * s
        stats_ref[0:1, :] = s
        stats_ref[1:2, :] = c

    s = stats_ref[0:1, :]
    c = stats_ref[1:2, :]
    xb = xnum_ref[pl.ds(pid * BLK, BLK), :] * s + c

    h = jnp.dot(embs_ref[...], w1a_ref[...], preferred_element_type=jnp.float32)
    h = h + jnp.dot(xb, w1b_ref[...], preferred_element_type=jnp.float32)
    h = jnp.maximum(h + b1_ref[...], 0.0)
    h2 = jnp.dot(h.astype(w2_ref.dtype), w2_ref[...],
                 preferred_element_type=jnp.float32)
    h2 = jnp.maximum(h2 + b2_ref[...], 0.0)
    o = jnp.dot(h2.astype(wo_ref.dtype), wo_ref[...],
                preferred_element_type=jnp.float32)
    out_ref[...] = jax.nn.sigmoid(o + bo_ref[...])


def kernel(x_cat, x_num, emb_tables, gamma, beta, W1, b1, W2, b2, Wout, bout):
    B, F = x_cat.shape
    _, V, D = emb_tables.shape
    NN = x_num.shape[1]
    H1 = W1.shape[1]
    H2 = W2.shape[1]
    ED = F * D

    embs = _sc_gather(x_cat.reshape(-1).astype(jnp.int32),
                      emb_tables.astype(jnp.bfloat16).reshape(F * V, D),
                      F, V, D)
    embs = embs.reshape(B, ED)

    NP = 16
    x_num_p = jnp.pad(x_num, ((0, 0), (0, NP - NN)))
    gamma_p = jnp.pad(gamma, (0, NP - NN)).reshape(1, NP)
    beta_p = jnp.pad(beta, (0, NP - NN)).reshape(1, NP)
    W1a = W1[:ED].astype(jnp.bfloat16)
    W1b = jnp.pad(W1[ED:], ((0, NP - NN), (0, 0)))
    W2b = W2.astype(jnp.bfloat16)
    Woutb = Wout.astype(jnp.bfloat16)

    BLK = 1024
    grid = (B // BLK,)
    out = pl.pallas_call(
        functools.partial(_mlp_body, BLK),
        grid=grid,
        in_specs=[
            pl.BlockSpec((B, NP), lambda i: (0, 0)),
            pl.BlockSpec((BLK, ED), lambda i: (i, 0)),
            pl.BlockSpec((ED, H1), lambda i: (0, 0)),
            pl.BlockSpec((NP, H1), lambda i: (0, 0)),
            pl.BlockSpec((1, H1), lambda i: (0, 0)),
            pl.BlockSpec((H1, H2), lambda i: (0, 0)),
            pl.BlockSpec((1, H2), lambda i: (0, 0)),
            pl.BlockSpec((H2, 1), lambda i: (0, 0)),
            pl.BlockSpec((1, 1), lambda i: (0, 0)),
            pl.BlockSpec((1, NP), lambda i: (0, 0)),
            pl.BlockSpec((1, NP), lambda i: (0, 0)),
        ],
        out_specs=pl.BlockSpec((BLK, 1), lambda i: (i, 0)),
        out_shape=jax.ShapeDtypeStruct((B, 1), jnp.float32),
        scratch_shapes=[pltpu.VMEM((8, NP), jnp.float32)],
    )(x_num_p, embs, W1a, W1b, b1.reshape(1, H1), W2b, b2.reshape(1, H2),
      Woutb, bout.reshape(1, 1), gamma_p, beta_p)
    return out


# speedup vs baseline: 1.6556x; 1.6556x over previous
import functools

import jax
import jax.numpy as jnp
from jax import lax
from jax.experimental import pallas as pl
from jax.experimental.pallas import tpu as pltpu
from jax.experimental.pallas import tpu_sc as plsc

_NC = 2
_NS = 16
_NW = _NC * _NS
_LANES = 16
_DP = 64


def _sc_gather(x_cat_flat, table_flat, F, V):
    R = x_cat_flat.shape[0]
    dt = table_flat.dtype
    r_per_w = R // _NW
    CH = 832
    n_ch = r_per_w // CH
    n_vec = r_per_w // _LANES

    mesh = plsc.VectorSubcoreMesh(
        core_axis_name="c", subcore_axis_name="s",
        num_cores=_NC, num_subcores=_NS)

    @functools.partial(
        pl.kernel,
        out_type=jax.ShapeDtypeStruct((R, _DP), dt),
        mesh=mesh,
        scratch_types=[
            pltpu.VMEM((r_per_w,), jnp.int32),
            pltpu.VMEM((CH, _DP), dt),
            pltpu.VMEM((CH, _DP), dt),
            pltpu.SemaphoreType.DMA,
            pltpu.SemaphoreType.DMA,
        ],
        compiler_params=pltpu.CompilerParams(use_tc_tiling_on_sc=False),
    )
    def k(xc_hbm, tab_hbm, out_hbm, idx_v, rows_a, rows_b, sem_a, sem_b):
        wid = lax.axis_index("s") * _NC + lax.axis_index("c")
        base = wid * r_per_w
        pltpu.sync_copy(xc_hbm.at[pl.ds(base * 1, r_per_w)], idx_v)

        lane = lax.iota(jnp.int32, _LANES)

        def idx_body(i, carry):
            p = i * _LANES + lane
            f = lax.rem(p, F)
            sl = pl.ds(i * _LANES, _LANES)
            idx_v[sl] = idx_v[sl] + f * V
            return carry

        lax.fori_loop(0, n_vec, idx_body, 0, unroll=4)

        bufs = (rows_a, rows_b)
        sems = (sem_a, sem_b)

        def fire(c, buf, sem):
            return pltpu.async_copy(
                tab_hbm.at[idx_v.at[pl.ds(c * CH, CH)]], buf, sem)

        fire(0, bufs[0], sems[0]).wait()
        for c in range(n_ch):
            if c + 1 < n_ch:
                nxt = fire(c + 1, bufs[(c + 1) % 2], sems[(c + 1) % 2])
            pltpu.sync_copy(bufs[c % 2], out_hbm.at[pl.ds(base + c * CH, CH)])
            if c + 1 < n_ch:
                nxt.wait()

    return k(x_cat_flat, table_flat)


def _stats_body(x_ref, g_ref, b_ref, o_ref):
    x = x_ref[...]
    n = x.shape[0]
    mu = jnp.sum(x, axis=0, keepdims=True) * (1.0 / n)
    var = jnp.sum(x * x, axis=0, keepdims=True) * (1.0 / n) - mu * mu
    s = g_ref[...] * lax.rsqrt(var + 1e-5)
    o_ref[0:1, :] = s
    o_ref[1:2, :] = b_ref[...] - mu * s


def _mlp_body(xnum_ref, sc_ref, embs_ref, w1a_ref, w1b_ref, b1_ref, w2_ref,
              b2_ref, wo_ref, bo_ref, out_ref):
    xb = xnum_ref[...] * sc_ref[0:1, :] + sc_ref[1:2, :]
    e = embs_ref[...].astype(w1a_ref.dtype)
    h = jnp.dot(e, w1a_ref[...], preferred_element_type=jnp.float32)
    h = h + jnp.dot(xb, w1b_ref[...], preferred_element_type=jnp.float32)
    h = jnp.maximum(h + b1_ref[...], 0.0)
    h2 = jnp.dot(h.astype(w2_ref.dtype), w2_ref[...],
                 preferred_element_type=jnp.float32)
    h2 = jnp.maximum(h2 + b2_ref[...], 0.0)
    o = jnp.dot(h2.astype(wo_ref.dtype), wo_ref[...],
                preferred_element_type=jnp.float32)
    out_ref[...] = jax.nn.sigmoid(o + bo_ref[...])


def kernel(x_cat, x_num, emb_tables, gamma, beta, W1, b1, W2, b2, Wout, bout):
    B, F = x_cat.shape
    _, V, D = emb_tables.shape
    NN = x_num.shape[1]
    H1 = W1.shape[1]
    H2 = W2.shape[1]
    ED = F * D
    EDP = F * _DP

    NP = 16
    x_num_p = jnp.pad(x_num, ((0, 0), (0, NP - NN)))
    gamma_p = jnp.pad(gamma, (0, NP - NN)).reshape(1, NP)
    beta_p = jnp.pad(beta, (0, NP - NN)).reshape(1, NP)
    W1a = jnp.pad(W1[:ED].reshape(F, D, H1),
                  ((0, 0), (0, _DP - D), (0, 0))).reshape(EDP, H1)
    W1a = W1a.astype(jnp.bfloat16)
    W1b = jnp.pad(W1[ED:], ((0, NP - NN), (0, 0)))
    W2b = W2.astype(jnp.bfloat16)
    Woutb = Wout.astype(jnp.bfloat16)
    tab = jnp.pad(emb_tables, ((0, 0), (0, 0), (0, _DP - D))).reshape(
        F * V, _DP)

    stats = pl.pallas_call(
        _stats_body,
        out_shape=jax.ShapeDtypeStruct((2, NP), jnp.float32),
    )(x_num_p, gamma_p, beta_p)

    NSPLIT = 2
    BP = B // NSPLIT
    BLK = 1024
    xc_flat = x_cat.reshape(-1).astype(jnp.int32)

    embs_parts = [
        _sc_gather(lax.slice_in_dim(xc_flat, k * BP * F, (k + 1) * BP * F),
                   tab, F, V).reshape(BP, EDP)
        for k in range(NSPLIT)
    ]

    mlp = pl.pallas_call(
        _mlp_body,
        grid=(BP // BLK,),
        in_specs=[
            pl.BlockSpec((BLK, NP), lambda i: (i, 0)),
            pl.BlockSpec((2, NP), lambda i: (0, 0)),
            pl.BlockSpec((BLK, EDP), lambda i: (i, 0)),
            pl.BlockSpec((EDP, H1), lambda i: (0, 0)),
            pl.BlockSpec((NP, H1), lambda i: (0, 0)),
            pl.BlockSpec((1, H1), lambda i: (0, 0)),
            pl.BlockSpec((H1, H2), lambda i: (0, 0)),
            pl.BlockSpec((1, H2), lambda i: (0, 0)),
            pl.BlockSpec((H2, 1), lambda i: (0, 0)),
            pl.BlockSpec((1, 1), lambda i: (0, 0)),
        ],
        out_specs=pl.BlockSpec((BLK, 1), lambda i: (i, 0)),
        out_shape=jax.ShapeDtypeStruct((BP, 1), jnp.float32),
    )
    outs = [
        mlp(lax.slice_in_dim(x_num_p, k * BP, (k + 1) * BP), stats,
            embs_parts[k], W1a, W1b, b1.reshape(1, H1), W2b,
            b2.reshape(1, H2), Woutb, bout.reshape(1, 1))
        for k in range(NSPLIT)
    ]
    return jnp.concatenate(outs, axis=0) if NSPLIT > 1 else outs[0]
